# baseline (speedup 1.0000x reference)
import jax
import jax.numpy as jnp
from jax import lax
from jax.experimental import pallas as pl
from jax.experimental.pallas import tpu as pltpu
from jax.experimental.pallas import tpu_sc as plsc

TBL = 1000000
D = 64
MD = 128
BATCH = 4096
HIST = 50
B_TOTAL = BATCH * HIST

NC, NS = 2, 16
NW = NC * NS
B_PER_W = B_TOTAL // NW
CHUNK = 128
NCH = B_PER_W // CHUNK


def _proj_body(t_ref, w_ref, p_ref):
    pb = lax.dot_general(
        t_ref[...].astype(jnp.bfloat16), w_ref[...].astype(jnp.bfloat16),
        (((0,), (0,)), ((), ())),
        preferred_element_type=jnp.float32,
    )
    pr = pb.reshape(pb.shape[0] // 2, 2, MD)
    even = pr[:, 0, :]
    odd = pr[:, 1, :]
    e16 = lax.bitcast_convert_type(even.astype(jnp.bfloat16), jnp.uint16)
    o16 = lax.bitcast_convert_type(odd.astype(jnp.bfloat16), jnp.uint16)
    p_ref[...] = e16.astype(jnp.uint32) | (o16.astype(jnp.uint32) << 16)


_NCOL = 16384
_proj = pl.pallas_call(
    _proj_body,
    grid=(pl.cdiv(TBL, _NCOL),),
    in_specs=[
        pl.BlockSpec((D, _NCOL), lambda i: (0, i)),
        pl.BlockSpec((D, MD), lambda i: (0, 0)),
    ],
    out_specs=pl.BlockSpec((_NCOL // 2, MD), lambda i: (i, 0)),
    out_shape=jax.ShapeDtypeStruct((TBL // 2, MD), jnp.uint32),
)


def _gather_body(idx2_hbm, par_hbm, p_hbm, out_hbm,
                 idx2_v, par_v, raw_v, out_v, sem_g, sem_w):
    wid = lax.axis_index("s") * NC + lax.axis_index("c")
    base = wid * B_PER_W
    pltpu.sync_copy(idx2_hbm.at[wid], idx2_v)
    pltpu.sync_copy(par_hbm.at[wid], par_v)
    pltpu.async_copy(p_hbm.at[idx2_v.at[0]], raw_v.at[0], sem_g)
    pltpu.async_copy(p_hbm.at[idx2_v.at[1]], raw_v.at[1], sem_g)

    def body(j, carry):
        s = j % 2
        pltpu.make_async_copy(
            p_hbm.at[idx2_v.at[0]], raw_v.at[0], sem_g
        ).wait()

        @pl.when(j >= 2)
        def _():
            pltpu.make_async_copy(
                out_v.at[0], out_hbm.at[pl.ds(base, CHUNK)], sem_w
            ).wait()

        def grp(g, c2):
            pv = par_v[j, pl.ds(g * 16, 16)]
            for i in range(16):
                r = g * 16 + i
                p = pv[i]
                sh = ((1 - p) * 16).astype(jnp.uint32)
                msk = jnp.where(p == 1, jnp.uint32(0xFFFF0000),
                                jnp.uint32(0xFFFFFFFF))
                for c in range(MD // 16):
                    w = raw_v[s, r, pl.ds(16 * c, 16)]
                    bits = (w << sh) & msk
                    out_v[s, r, pl.ds(16 * c, 16)] = plsc.bitcast(
                        bits, jnp.float32)
            return c2

        lax.fori_loop(0, CHUNK // 16, grp, 0)

        @pl.when(j + 2 < NCH)
        def _():
            pltpu.async_copy(p_hbm.at[idx2_v.at[j + 2]], raw_v.at[s], sem_g)

        pltpu.async_copy(
            out_v.at[s], out_hbm.at[pl.ds(base + j * CHUNK, CHUNK)], sem_w
        )
        return carry

    lax.fori_loop(0, NCH, body, 0)
    pltpu.make_async_copy(out_v.at[0], out_hbm.at[pl.ds(base, CHUNK)], sem_w).wait()
    pltpu.make_async_copy(out_v.at[0], out_hbm.at[pl.ds(base, CHUNK)], sem_w).wait()


_gather = pl.kernel(
    _gather_body,
    out_type=jax.ShapeDtypeStruct((B_TOTAL, MD), jnp.float32),
    mesh=plsc.VectorSubcoreMesh(
        core_axis_name="c", subcore_axis_name="s", num_cores=NC, num_subcores=NS
    ),
    scratch_types=[
        pltpu.VMEM((NCH, CHUNK), jnp.int32),
        pltpu.VMEM((NCH, CHUNK), jnp.int32),
        pltpu.VMEM((2, CHUNK, MD), jnp.uint32),
        pltpu.VMEM((2, CHUNK, MD), jnp.float32),
        pltpu.SemaphoreType.DMA,
        pltpu.SemaphoreType.DMA,
    ],
    compiler_params=pltpu.CompilerParams(
        use_tc_tiling_on_sc=True, needs_layout_passes=False
    ),
)


def kernel(input, table, W):
    p = _proj(table.T, W.T)
    idx = jnp.transpose(input).reshape(NW, NCH, CHUNK).astype(jnp.int32)
    out_flat = _gather(idx >> 1, idx & 1, p)
    return jnp.transpose(out_flat.reshape(HIST, BATCH, MD), (1, 0, 2))


# speedup vs baseline: 2.4772x; 2.4772x over previous
import jax
import jax.numpy as jnp
from jax import lax
from jax.experimental import pallas as pl
from jax.experimental.pallas import tpu as pltpu
from jax.experimental.pallas import tpu_sc as plsc

TBL = 1000000
D = 64
MD = 128
BATCH = 4096
HIST = 50
B_TOTAL = BATCH * HIST

NC, NS = 2, 16
NW = NC * NS
B_PER_W = B_TOTAL // NW
CHUNK = 128
NCH = B_PER_W // CHUNK


_NCOL = 8192
_NBLK = 62
H = _NBLK * _NCOL
_TBLK = pl.cdiv(TBL, _NCOL) - 1


def _rne16(b):
    return (b + jnp.uint32(0x7FFF) + ((b >> 16) & jnp.uint32(1)))


def _proj_body(tlo_ref, thi_ref, w_ref, p_ref):
    w16 = w_ref[...].astype(jnp.bfloat16)
    dims = (((0,), (0,)), ((), ()))
    lo = lax.dot_general(tlo_ref[...].astype(jnp.bfloat16), w16, dims,
                         preferred_element_type=jnp.float32)
    hi = lax.dot_general(thi_ref[...].astype(jnp.bfloat16), w16, dims,
                         preferred_element_type=jnp.float32)
    lb = _rne16(lax.bitcast_convert_type(lo, jnp.uint32)) >> 16
    hb = _rne16(lax.bitcast_convert_type(hi, jnp.uint32)) & jnp.uint32(0xFFFF0000)
    p_ref[...] = lb | hb


_proj = pl.pallas_call(
    _proj_body,
    grid=(_NBLK,),
    in_specs=[
        pl.BlockSpec((D, _NCOL), lambda i: (0, i)),
        pl.BlockSpec((D, _NCOL), lambda i: (0, jnp.minimum(i + _NBLK, _TBLK))),
        pl.BlockSpec((D, MD), lambda i: (0, 0)),
    ],
    out_specs=pl.BlockSpec((_NCOL, MD), lambda i: (i, 0)),
    out_shape=jax.ShapeDtypeStruct((H, MD), jnp.uint32),
)


def _gather_body(idx2_hbm, par_hbm, p_hbm, out_hbm,
                 idx2_v, par_v, raw_v, out_v, sem_g, sem_w):
    wid = lax.axis_index("s") * NC + lax.axis_index("c")
    base = wid * B_PER_W
    pltpu.sync_copy(idx2_hbm.at[wid], idx2_v)
    pltpu.sync_copy(par_hbm.at[wid], par_v)
    pltpu.async_copy(p_hbm.at[idx2_v.at[0]], raw_v.at[0], sem_g)
    pltpu.async_copy(p_hbm.at[idx2_v.at[1]], raw_v.at[1], sem_g)

    def body(j, carry):
        s = j % 2
        pltpu.make_async_copy(
            p_hbm.at[idx2_v.at[0]], raw_v.at[0], sem_g
        ).wait()

        @pl.when(j >= 2)
        def _():
            pltpu.make_async_copy(
                out_v.at[0], out_hbm.at[pl.ds(base, CHUNK)], sem_w
            ).wait()

        def grp(g, c2):
            pv = par_v[j, pl.ds(g * 16, 16)]
            for i in range(16):
                r = g * 16 + i
                p = pv[i]
                sh = ((1 - p) * 16).astype(jnp.uint32)
                msk = jnp.where(p == 1, jnp.uint32(0xFFFF0000),
                                jnp.uint32(0xFFFFFFFF))
                for c in range(MD // 16):
                    w = raw_v[s, r, pl.ds(16 * c, 16)]
                    bits = (w << sh) & msk
                    out_v[s, r, pl.ds(16 * c, 16)] = plsc.bitcast(
                        bits, jnp.float32)
            return c2

        lax.fori_loop(0, CHUNK // 16, grp, 0)

        @pl.when(j + 2 < NCH)
        def _():
            pltpu.async_copy(p_hbm.at[idx2_v.at[j + 2]], raw_v.at[s], sem_g)

        pltpu.async_copy(
            out_v.at[s], out_hbm.at[pl.ds(base + j * CHUNK, CHUNK)], sem_w
        )
        return carry

    lax.fori_loop(0, NCH, body, 0)
    pltpu.make_async_copy(out_v.at[0], out_hbm.at[pl.ds(base, CHUNK)], sem_w).wait()
    pltpu.make_async_copy(out_v.at[0], out_hbm.at[pl.ds(base, CHUNK)], sem_w).wait()


_gather = pl.kernel(
    _gather_body,
    out_type=jax.ShapeDtypeStruct((B_TOTAL, MD), jnp.float32),
    mesh=plsc.VectorSubcoreMesh(
        core_axis_name="c", subcore_axis_name="s", num_cores=NC, num_subcores=NS
    ),
    scratch_types=[
        pltpu.VMEM((NCH, CHUNK), jnp.int32),
        pltpu.VMEM((NCH, CHUNK), jnp.int32),
        pltpu.VMEM((2, CHUNK, MD), jnp.uint32),
        pltpu.VMEM((2, CHUNK, MD), jnp.float32),
        pltpu.SemaphoreType.DMA,
        pltpu.SemaphoreType.DMA,
    ],
    compiler_params=pltpu.CompilerParams(
        use_tc_tiling_on_sc=True, needs_layout_passes=False
    ),
)


def kernel(input, table, W):
    tT = table.T
    p = _proj(tT, tT, W.T)
    idx = jnp.transpose(input).reshape(NW, NCH, CHUNK).astype(jnp.int32)
    hi = (idx >= H).astype(jnp.int32)
    out_flat = _gather(idx - hi * H, hi, p)
    return jnp.transpose(out_flat.reshape(HIST, BATCH, MD), (1, 0, 2))
